# Initial kernel scaffold; baseline (speedup 1.0000x reference)
import jax
import jax.numpy as jnp
from jax.experimental import pallas as pl


def kernel(x, token_table, pos_table):
    raise NotImplementedError("write your pallas kernel here")


# speedup vs baseline: 3.3387x; 3.3387x over previous
import functools

import jax
import jax.numpy as jnp
from jax import lax
from jax.experimental import pallas as pl
from jax.experimental.pallas import tpu as pltpu
from jax.experimental.pallas import tpu_sc as plsc

_VOCAB = 100000
_MAXLEN = 200
_EMBED = 64
_BATCH = 4096

_NC = 2
_NS = 16
_NW = _NC * _NS
_ROWS_PER_W = _BATCH // _NW


def _emb_body(x_hbm, tok_hbm, pos_hbm, out_hbm, idx_v, pos_v, row_v, sem1, sem2):
    wid = lax.axis_index("s") * _NC + lax.axis_index("c")
    base_row = wid * _ROWS_PER_W

    pltpu.sync_copy(x_hbm.at[pl.ds(base_row * _MAXLEN, _ROWS_PER_W * _MAXLEN)],
                    idx_v)
    pltpu.sync_copy(pos_hbm, pos_v)

    def row_fn(r, carry):
        off = pl.multiple_of(r * _MAXLEN, 8)
        cp1 = pltpu.async_copy(tok_hbm.at[idx_v.at[pl.ds(off, 128)]],
                               row_v.at[pl.ds(0, 128)], sem1)
        cp2 = pltpu.async_copy(tok_hbm.at[idx_v.at[pl.ds(off + 128, 72)]],
                               row_v.at[pl.ds(128, 72)], sem2)
        cp1.wait()
        cp2.wait()

        def add_fn(i, c):
            for j in range(_EMBED // 16):
                sl = pl.ds(j * 16, 16)
                plsc.addupdate(row_v.at[i, sl], pos_v[i, sl])
            return c

        lax.fori_loop(0, _MAXLEN, add_fn, 0)

        pltpu.sync_copy(row_v, out_hbm.at[base_row + r])
        return carry

    lax.fori_loop(0, _ROWS_PER_W, row_fn, 0)


@jax.jit
def kernel(x, token_table, pos_table):
    mesh = plsc.VectorSubcoreMesh(core_axis_name="c", subcore_axis_name="s")
    emb = pl.kernel(
        _emb_body,
        out_type=jax.ShapeDtypeStruct((_BATCH, _MAXLEN, _EMBED), jnp.float32),
        mesh=mesh,
        scratch_types=[
            pltpu.VMEM((_ROWS_PER_W * _MAXLEN,), jnp.int32),
            pltpu.VMEM((_MAXLEN, _EMBED), jnp.float32),
            pltpu.VMEM((_MAXLEN, _EMBED), jnp.float32),
            pltpu.SemaphoreType.DMA,
            pltpu.SemaphoreType.DMA,
        ],
        compiler_params=pltpu.CompilerParams(use_tc_tiling_on_sc=False),
    )
    return emb(x.reshape(-1).astype(jnp.int32), token_table, pos_table)


# speedup vs baseline: 4.2316x; 1.2674x over previous
import jax
import jax.numpy as jnp
from jax import lax
from jax.experimental import pallas as pl
from jax.experimental.pallas import tpu as pltpu
from jax.experimental.pallas import tpu_sc as plsc

_VOCAB = 100000
_MAXLEN = 200
_EMBED = 64
_BATCH = 4096

_NC = 2
_NS = 16
_NW = _NC * _NS
_ROWS_PER_W = _BATCH // _NW
_NBUF = 4


def _emb_body(x_hbm, tok_hbm, pos_hbm, out_hbm, idx_v, pos_v, rows_v,
              gs0, gs1, gs2, gs3, ss0, ss1, ss2, ss3):
    gsem = (gs0, gs1, gs2, gs3)
    ssem = (ss0, ss1, ss2, ss3)
    wid = lax.axis_index("s") * _NC + lax.axis_index("c")
    base_row = wid * _ROWS_PER_W

    pltpu.sync_copy(x_hbm.at[pl.ds(base_row * _MAXLEN, _ROWS_PER_W * _MAXLEN)],
                    idx_v)
    pltpu.sync_copy(pos_hbm, pos_v)

    def g_start(r, b):
        off = pl.multiple_of(r * _MAXLEN, 8)
        pltpu.async_copy(tok_hbm.at[idx_v.at[pl.ds(off, 128)]],
                         rows_v.at[b, pl.ds(0, 128)], gsem[b])
        pltpu.async_copy(tok_hbm.at[idx_v.at[pl.ds(off + 128, 72)]],
                         rows_v.at[b, pl.ds(128, 72)], gsem[b])

    def g_wait(b):
        pltpu.make_async_copy(tok_hbm.at[idx_v.at[pl.ds(0, 128)]],
                              rows_v.at[b, pl.ds(0, 128)], gsem[b]).wait()
        pltpu.make_async_copy(tok_hbm.at[idx_v.at[pl.ds(128, 72)]],
                              rows_v.at[b, pl.ds(128, 72)], gsem[b]).wait()

    def s_start(r, b):
        pltpu.async_copy(rows_v.at[b], out_hbm.at[base_row + r], ssem[b])

    def s_wait(b):
        pltpu.make_async_copy(rows_v.at[b], out_hbm.at[base_row],
                              ssem[b]).wait()

    def add_pos(b):
        def add_fn(i, c):
            row4 = pl.multiple_of(i * 4, 4)
            for k in range(4):
                for j in range(_EMBED // 16):
                    sl = pl.ds(j * 16, 16)
                    plsc.addupdate(rows_v.at[b, row4 + k, sl],
                                   pos_v[row4 + k, sl])
            return c

        lax.fori_loop(0, _MAXLEN // 4, add_fn, 0)

    g_start(0, 0)
    g_start(1, 1)

    for r in (0, 1):
        b = r % _NBUF
        g_start(r + 2, (r + 2) % _NBUF)
        g_wait(b)
        add_pos(b)
        s_start(r, b)

    def outer(g, c):
        r0 = 2 + g * _NBUF
        for b2 in range(_NBUF):
            r = r0 + b2
            b = (2 + b2) % _NBUF
            bo = b2
            s_wait(bo)
            g_start(r + 2, bo)
            g_wait(b)
            add_pos(b)
            s_start(r, b)
        return c

    lax.fori_loop(0, (_ROWS_PER_W - _NBUF) // _NBUF, outer, 0)

    for r in (_ROWS_PER_W - 2, _ROWS_PER_W - 1):
        b = r % _NBUF
        s_wait((r + 2) % _NBUF)
        g_wait(b)
        add_pos(b)
        s_start(r, b)

    s_wait(2)
    s_wait(3)


@jax.jit
def kernel(x, token_table, pos_table):
    mesh = plsc.VectorSubcoreMesh(core_axis_name="c", subcore_axis_name="s")
    emb = pl.kernel(
        _emb_body,
        out_type=jax.ShapeDtypeStruct((_BATCH, _MAXLEN, _EMBED), jnp.float32),
        mesh=mesh,
        scratch_types=[
            pltpu.VMEM((_ROWS_PER_W * _MAXLEN,), jnp.int32),
            pltpu.VMEM((_MAXLEN, _EMBED), jnp.float32),
            pltpu.VMEM((_NBUF, _MAXLEN, _EMBED), jnp.float32),
        ] + [pltpu.SemaphoreType.DMA] * (2 * _NBUF),
        compiler_params=pltpu.CompilerParams(use_tc_tiling_on_sc=False),
    )
    return emb(x.reshape(-1).astype(jnp.int32), token_table, pos_table)
